# baseline (speedup 1.0000x reference)
import functools

import jax
import jax.numpy as jnp
from jax.experimental import pallas as pl
from jax.experimental.pallas import tpu as pltpu

_NC = 4


def _moe_tile_kernel(x_ref, wg_ref, bg_ref, w1_hbm, b1_ref, w2_hbm, b2_ref,
                     o_ref, w1_vmem, w2_vmem, sems, *, n_experts, hidden):
    i = pl.program_id(0)
    hc = hidden // _NC

    def _w1_copy(c):
        sl = pl.ds(c * hc, hc)
        return pltpu.make_async_copy(w1_hbm.at[:, sl], w1_vmem.at[:, sl],
                                     sems.at[0, c])

    def _w2_copy(c):
        sl = pl.ds(c * hc, hc)
        return pltpu.make_async_copy(w2_hbm.at[sl, :], w2_vmem.at[sl, :],
                                     sems.at[1, c])

    @pl.when(i == 0)
    def _issue():
        for c in range(_NC):
            _w1_copy(c).start()
            _w2_copy(c).start()

    x = x_ref[...]

    logits = jnp.dot(x, wg_ref[...], preferred_element_type=jnp.float32)
    logits = logits + bg_ref[...]
    v1 = jnp.max(logits, axis=-1, keepdims=True)
    idx = jax.lax.broadcasted_iota(jnp.int32, logits.shape, 1)
    am = jnp.min(jnp.where(logits == v1, idx, n_experts), axis=-1, keepdims=True)
    neg = jnp.float32(-jnp.inf)
    v2 = jnp.max(jnp.where(idx == am, neg, logits), axis=-1, keepdims=True)
    scale = 2.0 - jax.nn.sigmoid(v1 - v2)

    h = None
    for c in range(_NC):
        @pl.when(i == 0)
        def _wait(c=c):
            _w1_copy(c).wait()
            _w2_copy(c).wait()

        sl = pl.ds(c * hc, hc)
        u = jnp.dot(x, w1_vmem[:, sl], preferred_element_type=jnp.float32)
        u = jnp.maximum(u + b1_ref[:, sl], 0.0)
        hcur = jnp.dot(u, w2_vmem[sl, :], preferred_element_type=jnp.float32)
        h = hcur if h is None else h + hcur
    o_ref[...] = scale * (h + b2_ref[...])


def kernel(x, Wg, bg, W1, b1, W2, b2):
    B, N, D = x.shape
    T = B * N
    E = Wg.shape[1]
    H = W1.shape[1]
    xf = x.reshape(T, D)
    TILE = 512
    out = pl.pallas_call(
        functools.partial(_moe_tile_kernel, n_experts=E, hidden=H),
        grid=(T // TILE,),
        in_specs=[
            pl.BlockSpec((TILE, D), lambda i: (i, 0)),
            pl.BlockSpec((D, E), lambda i: (0, 0)),
            pl.BlockSpec((1, E), lambda i: (0, 0)),
            pl.BlockSpec(memory_space=pltpu.MemorySpace.HBM),
            pl.BlockSpec((1, H), lambda i: (0, 0)),
            pl.BlockSpec(memory_space=pltpu.MemorySpace.HBM),
            pl.BlockSpec((1, D), lambda i: (0, 0)),
        ],
        out_specs=pl.BlockSpec((TILE, D), lambda i: (i, 0)),
        out_shape=jax.ShapeDtypeStruct((T, D), jnp.float32),
        scratch_shapes=[
            pltpu.MemorySpace.VMEM((D, H), jnp.float32),
            pltpu.MemorySpace.VMEM((H, D), jnp.float32),
            pltpu.SemaphoreType.DMA((2, _NC)),
        ],
    )(xf, Wg, bg.reshape(1, E), W1, b1.reshape(1, H), W2, b2.reshape(1, D))
    return out.reshape(B, N, D)


# speedup vs baseline: 1.0787x; 1.0787x over previous
import functools

import jax
import jax.numpy as jnp
from jax.experimental import pallas as pl
from jax.experimental.pallas import tpu as pltpu

_NC = 4


def _moe_tile_kernel(x_ref, wg_ref, bg_ref, w1_hbm, b1_ref, w2_hbm, b2_ref,
                     o_ref, w1_vmem, w2_vmem, sems, *, n_experts, hidden):
    i = pl.program_id(0)
    hc = hidden // _NC

    def _w1_copy(c):
        sl = pl.ds(c * hc, hc)
        return pltpu.make_async_copy(w1_hbm.at[:, sl], w1_vmem.at[:, sl],
                                     sems.at[0, c])

    def _w2_copy(c):
        sl = pl.ds(c * hc, hc)
        return pltpu.make_async_copy(w2_hbm.at[sl, :], w2_vmem.at[sl, :],
                                     sems.at[1, c])

    @pl.when(i == 0)
    def _issue():
        for c in range(_NC):
            _w1_copy(c).start()
            _w2_copy(c).start()

    x = x_ref[...]

    logits = jnp.dot(x, wg_ref[...], preferred_element_type=jnp.float32)
    logits = logits + bg_ref[...]
    v1 = jnp.max(logits, axis=-1, keepdims=True)
    idx = jax.lax.broadcasted_iota(jnp.int32, logits.shape, 1)
    am = jnp.min(jnp.where(logits == v1, idx, n_experts), axis=-1, keepdims=True)
    neg = jnp.float32(-jnp.inf)
    v2 = jnp.max(jnp.where(idx == am, neg, logits), axis=-1, keepdims=True)
    scale = 2.0 - jax.nn.sigmoid(v1 - v2)

    @pl.when(i == 0)
    def _ffn_streamed():
        h = None
        for c in range(_NC):
            _w1_copy(c).wait()
            _w2_copy(c).wait()
            sl = pl.ds(c * hc, hc)
            u = jnp.dot(x, w1_vmem[:, sl], preferred_element_type=jnp.float32)
            u = jnp.maximum(u + b1_ref[:, sl], 0.0)
            hcur = jnp.dot(u, w2_vmem[sl, :], preferred_element_type=jnp.float32)
            h = hcur if h is None else h + hcur
        o_ref[...] = scale * (h + b2_ref[...])

    @pl.when(i > 0)
    def _ffn_resident():
        u = jnp.dot(x, w1_vmem[...], preferred_element_type=jnp.float32)
        u = jnp.maximum(u + b1_ref[...], 0.0)
        h = jnp.dot(u, w2_vmem[...], preferred_element_type=jnp.float32)
        o_ref[...] = scale * (h + b2_ref[...])


def kernel(x, Wg, bg, W1, b1, W2, b2):
    B, N, D = x.shape
    T = B * N
    E = Wg.shape[1]
    H = W1.shape[1]
    xf = x.reshape(T, D)
    TILE = 512
    out = pl.pallas_call(
        functools.partial(_moe_tile_kernel, n_experts=E, hidden=H),
        grid=(T // TILE,),
        in_specs=[
            pl.BlockSpec((TILE, D), lambda i: (i, 0)),
            pl.BlockSpec((D, E), lambda i: (0, 0)),
            pl.BlockSpec((1, E), lambda i: (0, 0)),
            pl.BlockSpec(memory_space=pltpu.MemorySpace.HBM),
            pl.BlockSpec((1, H), lambda i: (0, 0)),
            pl.BlockSpec(memory_space=pltpu.MemorySpace.HBM),
            pl.BlockSpec((1, D), lambda i: (0, 0)),
        ],
        out_specs=pl.BlockSpec((TILE, D), lambda i: (i, 0)),
        out_shape=jax.ShapeDtypeStruct((T, D), jnp.float32),
        scratch_shapes=[
            pltpu.MemorySpace.VMEM((D, H), jnp.float32),
            pltpu.MemorySpace.VMEM((H, D), jnp.float32),
            pltpu.SemaphoreType.DMA((2, _NC)),
        ],
    )(xf, Wg, bg.reshape(1, E), W1, b1.reshape(1, H), W2, b2.reshape(1, D))
    return out.reshape(B, N, D)
